# baseline (speedup 1.0000x reference)
import jax
import jax.numpy as jnp
from jax.experimental import pallas as pl
from jax.experimental.pallas import tpu as pltpu

INPUT_DIM = 1024
LATENT_DIM = 4096
K = 64
TM = 256
NCHUNK = 64
WINDOW = 1 << 16

INT_MIN = -(2**31)


def _count(mask):
    return jnp.sum(mask.astype(jnp.int32), axis=1, keepdims=True)


def _body(x_ref, we_ref, be_ref, wd_ref, bd_ref, sp_ref, rec_ref):
    lat = jax.lax.dot_general(
        x_ref[...], we_ref[...], (((1,), (1,)), ((), ())),
        preferred_element_type=jnp.float32,
    ) + be_ref[...]

    ikey = jax.lax.bitcast_convert_type(lat, jnp.int32)
    key = jnp.where(ikey < 0, ikey ^ jnp.int32(0x7FFFFFFF), ikey)

    key_hi = jax.lax.shift_right_arithmetic(key, 16).astype(jnp.int16)

    def count16(mask):
        return jnp.sum(mask.astype(jnp.int16), axis=1, keepdims=True)

    k16 = jnp.int16(K)
    lo0 = jnp.full((TM, 1), -32768, dtype=jnp.int32)
    hi0 = jnp.full((TM, 1), 32768, dtype=jnp.int32)

    def bis_hi(_, state):
        lo, hi = state
        mid = lo + jax.lax.shift_right_logical(hi - lo, 1)
        big = count16(key_hi >= mid.astype(jnp.int16)) >= k16
        return jnp.where(big, mid, lo), jnp.where(big, hi, mid)

    t_hi, _ = jax.lax.fori_loop(0, 16, bis_hi, (lo0, hi0))
    t_hi16 = t_hi.astype(jnp.int16)

    r = (K - count16(key_hi > t_hi16).astype(jnp.int32)).astype(jnp.int16)

    low_s = ((key & 0xFFFF) - 32768).astype(jnp.int16)
    z = jnp.where(key_hi == t_hi16, low_s, jnp.int16(-32768))

    def bis_lo(_, state):
        lo, hi = state
        mid = lo + jax.lax.shift_right_logical(hi - lo, 1)
        big = count16(z >= mid.astype(jnp.int16)) >= r
        return jnp.where(big, mid, lo), jnp.where(big, hi, mid)

    t_lo, _ = jax.lax.fori_loop(0, 16, bis_lo, (lo0, hi0))

    t = jax.lax.shift_left(t_hi, 16) | (t_lo + 32768)

    sparse = jnp.where(key >= t, lat, 0.0)
    sp_ref[...] = sparse

    rec = jax.lax.dot_general(
        sparse.astype(jnp.bfloat16), wd_ref[...], (((1,), (1,)), ((), ())),
        preferred_element_type=jnp.float32,
    ) + bd_ref[...]
    rec_ref[...] = rec


@jax.jit
def kernel(x, W_enc, b_enc, W_dec, b_dec):
    B = x.shape[0]
    grid = (B // TM,)
    out = pl.pallas_call(
        _body,
        grid=grid,
        in_specs=[
            pl.BlockSpec((TM, INPUT_DIM), lambda i: (i, 0)),
            pl.BlockSpec((LATENT_DIM, INPUT_DIM), lambda i: (0, 0)),
            pl.BlockSpec((1, LATENT_DIM), lambda i: (0, 0)),
            pl.BlockSpec((INPUT_DIM, LATENT_DIM), lambda i: (0, 0)),
            pl.BlockSpec((1, INPUT_DIM), lambda i: (0, 0)),
        ],
        out_specs=[
            pl.BlockSpec((TM, LATENT_DIM), lambda i: (i, 0)),
            pl.BlockSpec((TM, INPUT_DIM), lambda i: (i, 0)),
        ],
        out_shape=[
            jax.ShapeDtypeStruct((B, LATENT_DIM), jnp.float32),
            jax.ShapeDtypeStruct((B, INPUT_DIM), jnp.float32),
        ],
        compiler_params=pltpu.CompilerParams(
            vmem_limit_bytes=100 * 1024 * 1024,
        ),
    )(x, W_enc, b_enc.reshape(1, LATENT_DIM),
      W_dec.astype(jnp.bfloat16), b_dec.reshape(1, INPUT_DIM))
    sparse, recon = out
    return (recon, sparse)


# speedup vs baseline: 2.1698x; 2.1698x over previous
import jax
import jax.numpy as jnp
from jax.experimental import pallas as pl
from jax.experimental.pallas import tpu as pltpu

INPUT_DIM = 1024
LATENT_DIM = 4096
K = 64
TM = 256
NCHUNK = 64
WINDOW = 1 << 16

INT_MIN = -(2**31)


def _count(mask):
    return jnp.sum(mask.astype(jnp.int32), axis=1, keepdims=True)


def _body(x_ref, we_ref, be_ref, wd_ref, bd_ref, sp_ref, rec_ref):
    lat = jax.lax.dot_general(
        x_ref[...], we_ref[...], (((1,), (1,)), ((), ())),
        preferred_element_type=jnp.float32,
    ) + be_ref[...]

    ikey = jax.lax.bitcast_convert_type(lat, jnp.int32)
    key = jnp.where(ikey < 0, ikey ^ jnp.int32(0x7FFFFFFF), ikey)

    cnt0 = _count(key >= 0)
    t = jnp.where(cnt0 >= K, jnp.int32(0), jnp.int32(INT_MIN))
    for bit in range(30, -1, -1):
        cand = t + jnp.int32(1 << bit)
        t = jnp.where(_count(key >= cand) >= K, cand, t)

    sparse = jnp.where(key >= t, lat, 0.0)
    sp_ref[...] = sparse

    rec = jax.lax.dot_general(
        sparse.astype(jnp.bfloat16), wd_ref[...], (((1,), (1,)), ((), ())),
        preferred_element_type=jnp.float32,
    ) + bd_ref[...]
    rec_ref[...] = rec


@jax.jit
def kernel(x, W_enc, b_enc, W_dec, b_dec):
    B = x.shape[0]
    grid = (B // TM,)
    out = pl.pallas_call(
        _body,
        grid=grid,
        in_specs=[
            pl.BlockSpec((TM, INPUT_DIM), lambda i: (i, 0)),
            pl.BlockSpec((LATENT_DIM, INPUT_DIM), lambda i: (0, 0)),
            pl.BlockSpec((1, LATENT_DIM), lambda i: (0, 0)),
            pl.BlockSpec((INPUT_DIM, LATENT_DIM), lambda i: (0, 0)),
            pl.BlockSpec((1, INPUT_DIM), lambda i: (0, 0)),
        ],
        out_specs=[
            pl.BlockSpec((TM, LATENT_DIM), lambda i: (i, 0)),
            pl.BlockSpec((TM, INPUT_DIM), lambda i: (i, 0)),
        ],
        out_shape=[
            jax.ShapeDtypeStruct((B, LATENT_DIM), jnp.float32),
            jax.ShapeDtypeStruct((B, INPUT_DIM), jnp.float32),
        ],
        compiler_params=pltpu.CompilerParams(
            vmem_limit_bytes=100 * 1024 * 1024,
        ),
    )(x, W_enc, b_enc.reshape(1, LATENT_DIM),
      W_dec.astype(jnp.bfloat16), b_dec.reshape(1, INPUT_DIM))
    sparse, recon = out
    return (recon, sparse)
